# baseline (speedup 1.0000x reference)
import functools

import jax
import jax.numpy as jnp
from jax.experimental import pallas as pl
from jax.experimental.pallas import tpu as pltpu

B = 4
N = 4096
M = 4096
D = 6
DP = 8
TM = 2048
NMT = M // TM


def _body(x1_ref, x2_ref, sq1_ref, sq2_ref, d1_ref, i1_ref, d2_ref, i2_ref):
    mt = pl.program_id(1)
    a = x1_ref[0]
    b = x2_ref[0]
    cross = jax.lax.dot_general(
        a, b, (((1,), (1,)), ((), ())),
        preferred_element_type=jnp.float32)
    d = (sq1_ref[0] + sq2_ref[0]) + jnp.float32(-2.0) * cross

    m2 = jnp.min(d, axis=0, keepdims=True)
    iota0 = jax.lax.broadcasted_iota(jnp.int32, (N, TM), 0)
    i2 = jnp.min(jnp.where(d == m2, iota0, N), axis=0, keepdims=True)
    d2_ref[0] = m2
    i2_ref[0] = i2

    m1 = jnp.min(d, axis=1, keepdims=True)
    iota1 = jax.lax.broadcasted_iota(jnp.int32, (N, TM), 1)
    i1 = jnp.min(jnp.where(d == m1, iota1, TM), axis=1, keepdims=True)
    i1 = i1 + mt * TM

    @pl.when(mt == 0)
    def _init():
        d1_ref[0] = m1
        i1_ref[0] = i1

    @pl.when(mt != 0)
    def _merge():
        prev = d1_ref[0]
        better = m1 < prev
        d1_ref[0] = jnp.where(better, m1, prev)
        i1_ref[0] = jnp.where(better, i1, i1_ref[0])


@functools.partial(jax.jit, static_argnums=())
def kernel(input1, input2):
    x1 = jnp.pad(input1, ((0, 0), (0, 0), (0, DP - D)))
    x2 = jnp.pad(input2, ((0, 0), (0, 0), (0, DP - D)))
    sq1 = jnp.sum(input1 * input1, axis=-1)[:, :, None]
    sq2 = jnp.sum(input2 * input2, axis=-1)[:, None, :]

    grid = (B, NMT)
    out = pl.pallas_call(
        _body,
        grid=grid,
        compiler_params=pltpu.CompilerParams(
            dimension_semantics=("parallel", "arbitrary")),
        in_specs=[
            pl.BlockSpec((1, N, DP), lambda b, mt: (b, 0, 0)),
            pl.BlockSpec((1, TM, DP), lambda b, mt: (b, mt, 0)),
            pl.BlockSpec((1, N, 1), lambda b, mt: (b, 0, 0)),
            pl.BlockSpec((1, 1, TM), lambda b, mt: (b, 0, mt)),
        ],
        out_specs=[
            pl.BlockSpec((1, N, 1), lambda b, mt: (b, 0, 0)),
            pl.BlockSpec((1, N, 1), lambda b, mt: (b, 0, 0)),
            pl.BlockSpec((1, 1, TM), lambda b, mt: (b, 0, mt)),
            pl.BlockSpec((1, 1, TM), lambda b, mt: (b, 0, mt)),
        ],
        out_shape=[
            jax.ShapeDtypeStruct((B, N, 1), jnp.float32),
            jax.ShapeDtypeStruct((B, N, 1), jnp.int32),
            jax.ShapeDtypeStruct((B, 1, M), jnp.float32),
            jax.ShapeDtypeStruct((B, 1, M), jnp.int32),
        ],
    )(x1, x2, sq1, sq2)
    dist1 = out[0][:, :, 0]
    idx1 = out[1][:, :, 0]
    dist2 = out[2][:, 0, :]
    idx2 = out[3][:, 0, :]
    return (dist1, dist2, idx1, idx2)


# speedup vs baseline: 1.1290x; 1.1290x over previous
import functools

import jax
import jax.numpy as jnp
from jax.experimental import pallas as pl
from jax.experimental.pallas import tpu as pltpu

B = 4
N = 4096
M = 4096
D = 6
DP = 8
TM = 2048
NMT = M // TM


def _body(x1_ref, x2_ref, sq1_ref, sq2_ref, d1_ref, i1_ref, d2_ref, i2_ref):
    mt = pl.program_id(1)
    a = x1_ref[0]
    b = x2_ref[0]
    cross = jax.lax.dot_general(
        a, b, (((1,), (1,)), ((), ())),
        preferred_element_type=jnp.float32)
    d = (sq1_ref[0] + sq2_ref[0]) + jnp.float32(-2.0) * cross

    val = d
    idx = None
    k = N
    while k > 8:
        h = k // 2
        va, vb = val[:h], val[h:]
        better = vb < va
        if idx is None:
            io = jax.lax.broadcasted_iota(jnp.int32, (h, TM), 0)
            idx = jnp.where(better, io + h, io)
        else:
            idx = jnp.where(better, idx[h:], idx[:h])
        val = jnp.where(better, vb, va)
        k = h
    m2 = jnp.min(val, axis=0, keepdims=True)
    i2 = jnp.min(jnp.where(val == m2, idx, N), axis=0, keepdims=True)
    d2_ref[0] = m2
    i2_ref[0] = i2

    cval = d
    cidx = None
    c = TM
    while c > 128:
        h = c // 2
        va, vb = cval[:, :h], cval[:, h:]
        better = vb < va
        if cidx is None:
            io = jax.lax.broadcasted_iota(jnp.int32, (N, h), 1)
            cidx = jnp.where(better, io + h, io)
        else:
            cidx = jnp.where(better, cidx[:, h:], cidx[:, :h])
        cval = jnp.where(better, vb, va)
        c = h
    m1 = jnp.min(cval, axis=1, keepdims=True)
    i1 = jnp.min(jnp.where(cval == m1, cidx, M), axis=1, keepdims=True)
    i1 = i1 + mt * TM

    @pl.when(mt == 0)
    def _init():
        d1_ref[0] = m1
        i1_ref[0] = i1

    @pl.when(mt != 0)
    def _merge():
        prev = d1_ref[0]
        better = m1 < prev
        d1_ref[0] = jnp.where(better, m1, prev)
        i1_ref[0] = jnp.where(better, i1, i1_ref[0])


@functools.partial(jax.jit, static_argnums=())
def kernel(input1, input2):
    x1 = jnp.pad(input1, ((0, 0), (0, 0), (0, DP - D)))
    x2 = jnp.pad(input2, ((0, 0), (0, 0), (0, DP - D)))
    sq1 = jnp.sum(input1 * input1, axis=-1)[:, :, None]
    sq2 = jnp.sum(input2 * input2, axis=-1)[:, None, :]

    grid = (B, NMT)
    out = pl.pallas_call(
        _body,
        grid=grid,
        compiler_params=pltpu.CompilerParams(
            dimension_semantics=("parallel", "arbitrary")),
        in_specs=[
            pl.BlockSpec((1, N, DP), lambda b, mt: (b, 0, 0)),
            pl.BlockSpec((1, TM, DP), lambda b, mt: (b, mt, 0)),
            pl.BlockSpec((1, N, 1), lambda b, mt: (b, 0, 0)),
            pl.BlockSpec((1, 1, TM), lambda b, mt: (b, 0, mt)),
        ],
        out_specs=[
            pl.BlockSpec((1, N, 1), lambda b, mt: (b, 0, 0)),
            pl.BlockSpec((1, N, 1), lambda b, mt: (b, 0, 0)),
            pl.BlockSpec((1, 1, TM), lambda b, mt: (b, 0, mt)),
            pl.BlockSpec((1, 1, TM), lambda b, mt: (b, 0, mt)),
        ],
        out_shape=[
            jax.ShapeDtypeStruct((B, N, 1), jnp.float32),
            jax.ShapeDtypeStruct((B, N, 1), jnp.int32),
            jax.ShapeDtypeStruct((B, 1, M), jnp.float32),
            jax.ShapeDtypeStruct((B, 1, M), jnp.int32),
        ],
    )(x1, x2, sq1, sq2)
    dist1 = out[0][:, :, 0]
    idx1 = out[1][:, :, 0]
    dist2 = out[2][:, 0, :]
    idx2 = out[3][:, 0, :]
    return (dist1, dist2, idx1, idx2)


# speedup vs baseline: 1.1468x; 1.0158x over previous
import functools

import jax
import jax.numpy as jnp
from jax.experimental import pallas as pl
from jax.experimental.pallas import tpu as pltpu

B = 4
N = 4096
M = 4096
D = 6
DP = 8
TM = 2048
NMT = M // TM


def _body(x1_ref, x2_ref, sq1_ref, sq2_ref, d1_ref, i1_ref, d2_ref, i2_ref):
    mt = pl.program_id(1)
    a = x1_ref[0]
    b = x2_ref[0]
    ncross = jax.lax.dot_general(
        a, b, (((1,), (1,)), ((), ())),
        preferred_element_type=jnp.float32)
    d = (sq1_ref[0] + sq2_ref[0]) + ncross

    val = d
    idx = None
    k = N
    while k > 8:
        h = k // 2
        va, vb = val[:h], val[h:]
        better = vb < va
        if idx is None:
            io = jax.lax.broadcasted_iota(jnp.int32, (h, TM), 0)
            idx = jnp.where(better, io + h, io)
        else:
            idx = jnp.where(better, idx[h:], idx[:h])
        val = jnp.where(better, vb, va)
        k = h
    m2 = jnp.min(val, axis=0, keepdims=True)
    i2 = jnp.min(jnp.where(val == m2, idx, N), axis=0, keepdims=True)
    d2_ref[0] = m2
    i2_ref[0] = i2

    cval = d
    cidx = None
    c = TM
    while c > 128:
        h = c // 2
        va, vb = cval[:, :h], cval[:, h:]
        better = vb < va
        if cidx is None:
            io = jax.lax.broadcasted_iota(jnp.int32, (N, h), 1)
            cidx = jnp.where(better, io + h, io)
        else:
            cidx = jnp.where(better, cidx[:, h:], cidx[:, :h])
        cval = jnp.where(better, vb, va)
        c = h
    m1 = jnp.min(cval, axis=1, keepdims=True)
    i1 = jnp.min(jnp.where(cval == m1, cidx, M), axis=1, keepdims=True)
    i1 = i1 + mt * TM

    @pl.when(mt == 0)
    def _init():
        d1_ref[0] = m1
        i1_ref[0] = i1

    @pl.when(mt != 0)
    def _merge():
        prev = d1_ref[0]
        better = m1 < prev
        d1_ref[0] = jnp.where(better, m1, prev)
        i1_ref[0] = jnp.where(better, i1, i1_ref[0])


@functools.partial(jax.jit, static_argnums=())
def kernel(input1, input2):
    x1 = jnp.pad(input1 * jnp.float32(-2.0), ((0, 0), (0, 0), (0, DP - D)))
    x2 = jnp.pad(input2, ((0, 0), (0, 0), (0, DP - D)))
    sq1 = jnp.sum(input1 * input1, axis=-1)[:, :, None]
    sq2 = jnp.sum(input2 * input2, axis=-1)[:, None, :]

    grid = (B, NMT)
    out = pl.pallas_call(
        _body,
        grid=grid,
        compiler_params=pltpu.CompilerParams(
            dimension_semantics=("parallel", "arbitrary")),
        in_specs=[
            pl.BlockSpec((1, N, DP), lambda b, mt: (b, 0, 0)),
            pl.BlockSpec((1, TM, DP), lambda b, mt: (b, mt, 0)),
            pl.BlockSpec((1, N, 1), lambda b, mt: (b, 0, 0)),
            pl.BlockSpec((1, 1, TM), lambda b, mt: (b, 0, mt)),
        ],
        out_specs=[
            pl.BlockSpec((1, N, 1), lambda b, mt: (b, 0, 0)),
            pl.BlockSpec((1, N, 1), lambda b, mt: (b, 0, 0)),
            pl.BlockSpec((1, 1, TM), lambda b, mt: (b, 0, mt)),
            pl.BlockSpec((1, 1, TM), lambda b, mt: (b, 0, mt)),
        ],
        out_shape=[
            jax.ShapeDtypeStruct((B, N, 1), jnp.float32),
            jax.ShapeDtypeStruct((B, N, 1), jnp.int32),
            jax.ShapeDtypeStruct((B, 1, M), jnp.float32),
            jax.ShapeDtypeStruct((B, 1, M), jnp.int32),
        ],
    )(x1, x2, sq1, sq2)
    dist1 = out[0][:, :, 0]
    idx1 = out[1][:, :, 0]
    dist2 = out[2][:, 0, :]
    idx2 = out[3][:, 0, :]
    return (dist1, dist2, idx1, idx2)
